# baseline (speedup 1.0000x reference)
import functools

import jax
import jax.numpy as jnp
from jax import lax
from jax.experimental import pallas as pl
from jax.experimental.pallas import tpu as pltpu
from jax.experimental.pallas import tpu_sc as plsc

NUM_SAMPLED = 100
VOCAB = 1000000
DIM = 32
BATCH = 4096

_PACK = 128 // DIM
_S_PAD = 256
_S_COLS = 128


def _log_uniform_prob(classes_f32, range_max):
    return (jnp.log(classes_f32 + 2.0) - jnp.log(classes_f32 + 1.0)) / jnp.log(
        range_max + 1.0
    )


def _make_sc_gather(n_lab, n_samp, nc, ns):
    nw = nc * ns
    lab_per_w = n_lab // nw
    samp_per_w = n_samp // nw
    mesh = plsc.VectorSubcoreMesh(core_axis_name="c", subcore_axis_name="s")

    @functools.partial(
        pl.kernel,
        mesh=mesh,
        out_type=jax.ShapeDtypeStruct((n_lab + n_samp, 128), jnp.float32),
        scratch_types=[
            pltpu.VMEM((lab_per_w,), jnp.int32),
            pltpu.VMEM((lab_per_w, 128), jnp.float32),
            pltpu.VMEM((samp_per_w,), jnp.int32),
            pltpu.VMEM((samp_per_w, 128), jnp.float32),
            pltpu.SemaphoreType.DMA,
            pltpu.SemaphoreType.DMA,
        ],
    )
    def sc_gather(table_hbm, idx_hbm, out_hbm, idx_l, rows_l, idx_s, rows_s,
                  sem_l, sem_s):
        wid = lax.axis_index("s") * nc + lax.axis_index("c")
        base_l = wid * lab_per_w
        base_s = n_lab + wid * samp_per_w
        pltpu.sync_copy(idx_hbm.at[pl.ds(base_l, lab_per_w)], idx_l)
        pltpu.sync_copy(idx_hbm.at[pl.ds(base_s, samp_per_w)], idx_s)
        g_l = pltpu.async_copy(table_hbm.at[idx_l], rows_l, sem_l)
        g_s = pltpu.async_copy(table_hbm.at[idx_s], rows_s, sem_s)
        g_l.wait()
        pltpu.sync_copy(rows_l, out_hbm.at[pl.ds(base_l, lab_per_w)])
        g_s.wait()
        pltpu.sync_copy(rows_s, out_hbm.at[pl.ds(base_s, samp_per_w)])

    return sc_gather


def _select_sub(packed, sub):
    out = packed[:, 0:DIM]
    for j in range(1, _PACK):
        out = jnp.where(sub == j, packed[:, j * DIM : (j + 1) * DIM], out)
    return out


def _tc_body(user_ref, truep_ref, sampp_ref, labels_ref, sidx_ref, slog_ref,
             ssub_ref, out_ref):
    u = user_ref[...]
    lab = labels_ref[...]
    sidx = sidx_ref[...]
    slog = slog_ref[...]

    tw = _select_sub(truep_ref[...], lab & (_PACK - 1))
    sw = _select_sub(sampp_ref[...], ssub_ref[...])

    lf = lab.astype(jnp.float32)
    p_true = _log_uniform_prob(lf, float(VOCAB))
    true_logit = (
        jnp.sum(u * tw, axis=1, keepdims=True)
        - jnp.log(NUM_SAMPLED * p_true)
    )

    s_logits = (
        lax.dot_general(u, sw, (((1,), (1,)), ((), ())),
                        preferred_element_type=jnp.float32)
        - slog
    )
    col = lax.broadcasted_iota(jnp.int32, (1, _S_COLS), 1)
    dead = (sidx == lab) | (col >= NUM_SAMPLED)
    s_logits = jnp.where(dead, jnp.float32(-1e9), s_logits)

    m = jnp.maximum(jnp.max(s_logits, axis=1, keepdims=True), true_logit)
    ssum = jnp.sum(jnp.exp(s_logits - m), axis=1, keepdims=True) + jnp.exp(
        true_logit - m
    )
    out_ref[...] = m + jnp.log(ssum) - true_logit


def kernel(item_embeddings, user_embeddings, label_idx, zero_bias):
    del zero_bias
    table = item_embeddings.reshape(VOCAB // _PACK, 128)
    user = user_embeddings.reshape(BATCH, DIM)
    labels = label_idx.reshape(BATCH).astype(jnp.int32)

    skey = jax.random.key(42)
    u01 = jax.random.uniform(skey, (NUM_SAMPLED,), dtype=jnp.float32)
    sampled = jnp.clip(
        (jnp.exp(u01 * jnp.log(VOCAB + 1.0)) - 1.0).astype(labels.dtype),
        0,
        VOCAB - 1,
    )
    p_samp = _log_uniform_prob(sampled.astype(jnp.float32), float(VOCAB))
    slog = jnp.zeros((1, _S_COLS), jnp.float32).at[0, :NUM_SAMPLED].set(
        jnp.log(NUM_SAMPLED * p_samp)
    )
    sidx = jnp.full((1, _S_COLS), -1, jnp.int32).at[0, :NUM_SAMPLED].set(sampled)
    ssub = (
        jnp.zeros((_S_COLS, 1), jnp.int32)
        .at[:NUM_SAMPLED, 0]
        .set(sampled & (_PACK - 1))
    )

    info = plsc.get_sparse_core_info()
    nc, ns = info.num_cores, info.num_subcores
    idx_all = (
        jnp.concatenate(
            [labels, jnp.zeros((_S_PAD,), jnp.int32).at[:NUM_SAMPLED].set(sampled)]
        )
        // _PACK
    )
    gathered = _make_sc_gather(BATCH, _S_PAD, nc, ns)(table, idx_all)
    true_p = gathered[:BATCH]
    samp_p = gathered[BATCH : BATCH + _S_COLS]

    loss = pl.pallas_call(
        _tc_body,
        out_shape=jax.ShapeDtypeStruct((BATCH, 1), jnp.float32),
    )(user, true_p, samp_p, labels.reshape(BATCH, 1), sidx, slog, ssub)
    return loss


# speedup vs baseline: 5.9664x; 5.9664x over previous
import functools

import jax
import jax.numpy as jnp
from jax import lax
from jax.experimental import pallas as pl
from jax.experimental.pallas import tpu as pltpu
from jax.experimental.pallas import tpu_sc as plsc

NUM_SAMPLED = 100
VOCAB = 1000000
DIM = 32
BATCH = 4096

_S_COLS = 128
_LANES = 128
_CH = 8


def _log_uniform_prob(classes_f32, range_max):
    return (jnp.log(classes_f32 + 2.0) - jnp.log(classes_f32 + 1.0)) / jnp.log(
        range_max + 1.0
    )


def _make_sc_gather(nc, ns):
    nw = nc * ns
    lab_per_w = BATCH // nw
    n_bufs = 2 * _CH
    mesh = plsc.VectorSubcoreMesh(core_axis_name="c", subcore_axis_name="s")

    @functools.partial(
        pl.kernel,
        mesh=mesh,
        compiler_params=pltpu.CompilerParams(needs_layout_passes=False),
        out_type=(
            jax.ShapeDtypeStruct((BATCH * DIM,), jnp.float32),
            jax.ShapeDtypeStruct((_S_COLS * DIM,), jnp.float32),
        ),
        scratch_types=[
            pltpu.VMEM((lab_per_w + 16,), jnp.int32),
            pltpu.VMEM((n_bufs * DIM, _LANES), jnp.float32),
            pltpu.VMEM((lab_per_w * DIM,), jnp.float32),
            pltpu.VMEM((_S_COLS * DIM,), jnp.float32),
            pltpu.SemaphoreType.DMA,
        ],
    )
    def sc_gather(table_t, labels_hbm, sidx_hbm, true_out, samp_out,
                  idx_vmem, ring, rows_l, rows_s, sem):
        cid = lax.axis_index("c")
        sid = lax.axis_index("s")
        wid = sid * nc + cid
        base_l = pl.multiple_of(wid * lab_per_w, 8)
        pltpu.sync_copy(
            labels_hbm.at[pl.ds(base_l, lab_per_w)],
            idx_vmem.at[pl.ds(0, lab_per_w)],
        )

        iota = jax.lax.iota(jnp.int32, 16)
        mask_one = iota == 0
        int_min = jnp.int32(-2147483648)

        def pick(vec, t):
            return jnp.max(jnp.where(iota == t, vec, int_min))

        def fire(labvec, t, buf):
            col = pick(labvec, t)
            cbase = pl.multiple_of((col // _LANES) * _LANES, _LANES)
            pltpu.async_copy(
                table_t.at[:, pl.ds(cbase, _LANES)],
                ring.at[pl.ds(buf * DIM, DIM)],
                sem,
            )

        def drain_one(buf):
            pltpu.make_async_copy(
                table_t.at[:, pl.ds(0, _LANES)],
                ring.at[pl.ds(buf * DIM, DIM)],
                sem,
            ).wait()

        def extract_chunk(labvec, rows, chunk, phase):
            lanevec = labvec - (labvec // _LANES) * _LANES
            for t in range(_CH):
                lvec = jnp.take(lanevec, jnp.full((16,), t, jnp.int32))
                svec = jnp.full((16,), (chunk * _CH + t) * DIM, jnp.int32)
                buf_row = (phase + t) * DIM
                for d in range(DIM):
                    vals = plsc.load_gather(ring.at[buf_row + d], [lvec])
                    plsc.store_scatter(rows, [svec + d], vals, mask=mask_one)

        def load_labvec(c):
            off = pl.multiple_of(c * _CH, 8)
            return idx_vmem[pl.ds(off, 16)]

        def gather_cols(rows, n_slots):
            n_chunks = n_slots // _CH

            def chunk_body(c, carry):
                phase = (c % 2) * _CH
                prev_phase = ((c - 1) % 2) * _CH
                labvec = load_labvec(c)
                for t in range(_CH):
                    fire(labvec, t, phase + t)

                @pl.when(c > 0)
                def _():
                    for t in range(_CH):
                        drain_one(prev_phase + t)
                    extract_chunk(load_labvec(c - 1), rows, c - 1, prev_phase)

                return carry

            lax.fori_loop(0, n_chunks, chunk_body, 0)
            last = n_chunks - 1
            last_phase = (last % 2) * _CH
            for t in range(_CH):
                drain_one(last_phase + t)
            extract_chunk(load_labvec(last), rows, last, last_phase)

        gather_cols(rows_l, lab_per_w)
        out_base = pl.multiple_of(wid * lab_per_w * DIM, 8)
        pltpu.sync_copy(rows_l, true_out.at[pl.ds(out_base, lab_per_w * DIM)])

        @pl.when(wid == 0)
        def _():
            pltpu.sync_copy(
                sidx_hbm.at[pl.ds(0, _S_COLS)], idx_vmem.at[pl.ds(0, _S_COLS)]
            )
            gather_cols(rows_s, _S_COLS)
            pltpu.sync_copy(rows_s, samp_out.at[pl.ds(0, _S_COLS * DIM)])

    return sc_gather


def _tc_body(user_ref, truew_ref, sampw_ref, labels_ref, sidx_ref, slog_ref,
             out_ref):
    u = user_ref[...]
    tw = truew_ref[...]
    sw = sampw_ref[...]
    lab = labels_ref[...]
    sidx = sidx_ref[...]
    slog = slog_ref[...]

    lf = lab.astype(jnp.float32)
    p_true = _log_uniform_prob(lf, float(VOCAB))
    true_logit = (
        jnp.sum(u * tw, axis=1, keepdims=True)
        - jnp.log(NUM_SAMPLED * p_true)
    )

    s_logits = (
        lax.dot_general(u, sw, (((1,), (1,)), ((), ())),
                        preferred_element_type=jnp.float32)
        - slog
    )
    col = lax.broadcasted_iota(jnp.int32, (1, _S_COLS), 1)
    dead = (sidx == lab) | (col >= NUM_SAMPLED)
    s_logits = jnp.where(dead, jnp.float32(-1e9), s_logits)

    m = jnp.maximum(jnp.max(s_logits, axis=1, keepdims=True), true_logit)
    ssum = jnp.sum(jnp.exp(s_logits - m), axis=1, keepdims=True) + jnp.exp(
        true_logit - m
    )
    out_ref[...] = m + jnp.log(ssum) - true_logit


def kernel(item_embeddings, user_embeddings, label_idx, zero_bias):
    del zero_bias
    table_t = jnp.squeeze(item_embeddings, axis=1).T
    user = user_embeddings.reshape(BATCH, DIM)
    labels = label_idx.reshape(BATCH).astype(jnp.int32)

    skey = jax.random.key(42)
    u01 = jax.random.uniform(skey, (NUM_SAMPLED,), dtype=jnp.float32)
    sampled = jnp.clip(
        (jnp.exp(u01 * jnp.log(VOCAB + 1.0)) - 1.0).astype(labels.dtype),
        0,
        VOCAB - 1,
    )
    p_samp = _log_uniform_prob(sampled.astype(jnp.float32), float(VOCAB))
    slog = jnp.zeros((1, _S_COLS), jnp.float32).at[0, :NUM_SAMPLED].set(
        jnp.log(NUM_SAMPLED * p_samp)
    )
    sidx = jnp.full((1, _S_COLS), -1, jnp.int32).at[0, :NUM_SAMPLED].set(sampled)
    pad_spread = (jnp.arange(_S_COLS, dtype=jnp.int32) * 3929) % VOCAB
    samp_gather_idx = pad_spread.at[:NUM_SAMPLED].set(sampled)

    info = plsc.get_sparse_core_info()
    nc, ns = info.num_cores, info.num_subcores
    true_f, samp_f = _make_sc_gather(nc, ns)(table_t, labels, samp_gather_idx)
    true_w = true_f.reshape(BATCH, DIM)
    samp_w = samp_f.reshape(_S_COLS, DIM)

    loss = pl.pallas_call(
        _tc_body,
        out_shape=jax.ShapeDtypeStruct((BATCH, 1), jnp.float32),
    )(user, true_w, samp_w, labels.reshape(BATCH, 1), sidx, slog)
    return loss


# speedup vs baseline: 7.9780x; 1.3372x over previous
import functools

import jax
import jax.numpy as jnp
from jax import lax
from jax.experimental import pallas as pl
from jax.experimental.pallas import tpu as pltpu
from jax.experimental.pallas import tpu_sc as plsc

NUM_SAMPLED = 100
VOCAB = 1000000
DIM = 32
BATCH = 4096

_S_COLS = 128
_LANES = 128
_CH = 8


def _log_uniform_prob(classes_f32, range_max):
    return (jnp.log(classes_f32 + 2.0) - jnp.log(classes_f32 + 1.0)) / jnp.log(
        range_max + 1.0
    )


def _make_sc_gather(nc, ns):
    nw = nc * ns
    lab_per_w = BATCH // nw
    n_bufs = 2 * _CH
    mesh = plsc.VectorSubcoreMesh(core_axis_name="c", subcore_axis_name="s")

    @functools.partial(
        pl.kernel,
        mesh=mesh,
        compiler_params=pltpu.CompilerParams(needs_layout_passes=False),
        out_type=(
            jax.ShapeDtypeStruct((BATCH * DIM,), jnp.float32),
            jax.ShapeDtypeStruct((_S_COLS * DIM,), jnp.float32),
        ),
        scratch_types=[
            pltpu.VMEM((lab_per_w + 16,), jnp.int32),
            pltpu.VMEM((n_bufs * DIM, _LANES), jnp.float32),
            pltpu.VMEM((lab_per_w * DIM,), jnp.float32),
            pltpu.VMEM((4 * DIM,), jnp.float32),
            pltpu.SemaphoreType.DMA,
        ],
    )
    def sc_gather(table_t, labels_hbm, sidx_hbm, true_out, samp_out,
                  idx_vmem, ring, rows_l, rows_s, sem):
        cid = lax.axis_index("c")
        sid = lax.axis_index("s")
        wid = sid * nc + cid
        base_l = pl.multiple_of(wid * lab_per_w, 8)
        pltpu.sync_copy(
            labels_hbm.at[pl.ds(base_l, lab_per_w)],
            idx_vmem.at[pl.ds(0, lab_per_w)],
        )

        iota = jax.lax.iota(jnp.int32, 16)
        mask_one = iota == 0
        int_min = jnp.int32(-2147483648)

        def pick(vec, t):
            return jnp.max(jnp.where(iota == t, vec, int_min))

        def fire(labvec, t, buf):
            col = pick(labvec, t)
            cbase = pl.multiple_of((col // _LANES) * _LANES, _LANES)
            pltpu.async_copy(
                table_t.at[:, pl.ds(cbase, _LANES)],
                ring.at[pl.ds(buf * DIM, DIM)],
                sem,
            )

        def drain_one(buf):
            pltpu.make_async_copy(
                table_t.at[:, pl.ds(0, _LANES)],
                ring.at[pl.ds(buf * DIM, DIM)],
                sem,
            ).wait()

        def extract_chunk(labvec, rows, chunk, phase):
            lanevec = labvec - (labvec // _LANES) * _LANES
            for t in range(_CH):
                lvec = jnp.take(lanevec, jnp.full((16,), t, jnp.int32))
                svec = jnp.full((16,), (chunk * _CH + t) * DIM, jnp.int32)
                buf_row = (phase + t) * DIM
                for d in range(DIM):
                    vals = plsc.load_gather(ring.at[buf_row + d], [lvec])
                    plsc.store_scatter(rows, [svec + d], vals, mask=mask_one)

        def load_labvec(c):
            off = pl.multiple_of(c * _CH, 8)
            return idx_vmem[pl.ds(off, 16)]

        def gather_cols(rows, n_slots):
            n_chunks = n_slots // _CH

            def chunk_body(c, carry):
                phase = (c % 2) * _CH
                prev_phase = ((c - 1) % 2) * _CH
                labvec = load_labvec(c)
                for t in range(_CH):
                    fire(labvec, t, phase + t)

                @pl.when(c > 0)
                def _():
                    for t in range(_CH):
                        drain_one(prev_phase + t)
                    extract_chunk(load_labvec(c - 1), rows, c - 1, prev_phase)

                return carry

            lax.fori_loop(0, n_chunks, chunk_body, 0)
            last = n_chunks - 1
            last_phase = (last % 2) * _CH
            for t in range(_CH):
                drain_one(last_phase + t)
            extract_chunk(load_labvec(last), rows, last, last_phase)

        gather_cols(rows_l, lab_per_w)
        out_base = pl.multiple_of(wid * lab_per_w * DIM, 8)
        pltpu.sync_copy(rows_l, true_out.at[pl.ds(out_base, lab_per_w * DIM)])

        sbase = pl.multiple_of((wid // 2) * 8, 8)
        pltpu.sync_copy(sidx_hbm.at[pl.ds(sbase, 8)], idx_vmem.at[pl.ds(0, 8)])
        svecb = idx_vmem[pl.ds(0, 16)]
        lanevec_s = svecb - (svecb // _LANES) * _LANES
        off = (wid % 2) * 4
        for t in range(4):
            fire(svecb, off + t, t)
        for t in range(4):
            drain_one(t)
        for t in range(4):
            lvec = jnp.take(lanevec_s, jnp.full((16,), off + t, jnp.int32))
            svec = jnp.full((16,), t * DIM, jnp.int32)
            for d in range(DIM):
                vals = plsc.load_gather(ring.at[t * DIM + d], [lvec])
                plsc.store_scatter(rows_s, [svec + d], vals, mask=mask_one)
        pltpu.sync_copy(
            rows_s,
            samp_out.at[pl.ds(pl.multiple_of(wid * (4 * DIM), 8), 4 * DIM)],
        )

    return sc_gather


def _tc_body(user_ref, truew_ref, sampw_ref, labels_ref, sidx_ref, slog_ref,
             out_ref):
    u = user_ref[...]
    tw = truew_ref[...]
    sw = sampw_ref[...]
    lab = labels_ref[...]
    sidx = sidx_ref[...]
    slog = slog_ref[...]

    lf = lab.astype(jnp.float32)
    p_true = _log_uniform_prob(lf, float(VOCAB))
    true_logit = (
        jnp.sum(u * tw, axis=1, keepdims=True)
        - jnp.log(NUM_SAMPLED * p_true)
    )

    s_logits = (
        lax.dot_general(u, sw, (((1,), (1,)), ((), ())),
                        preferred_element_type=jnp.float32)
        - slog
    )
    col = lax.broadcasted_iota(jnp.int32, (1, _S_COLS), 1)
    dead = (sidx == lab) | (col >= NUM_SAMPLED)
    s_logits = jnp.where(dead, jnp.float32(-1e9), s_logits)

    m = jnp.maximum(jnp.max(s_logits, axis=1, keepdims=True), true_logit)
    ssum = jnp.sum(jnp.exp(s_logits - m), axis=1, keepdims=True) + jnp.exp(
        true_logit - m
    )
    out_ref[...] = m + jnp.log(ssum) - true_logit


def kernel(item_embeddings, user_embeddings, label_idx, zero_bias):
    del zero_bias
    table_t = jnp.squeeze(item_embeddings, axis=1).T
    user = user_embeddings.reshape(BATCH, DIM)
    labels = label_idx.reshape(BATCH).astype(jnp.int32)

    skey = jax.random.key(42)
    u01 = jax.random.uniform(skey, (NUM_SAMPLED,), dtype=jnp.float32)
    sampled = jnp.clip(
        (jnp.exp(u01 * jnp.log(VOCAB + 1.0)) - 1.0).astype(labels.dtype),
        0,
        VOCAB - 1,
    )
    p_samp = _log_uniform_prob(sampled.astype(jnp.float32), float(VOCAB))
    slog = jnp.zeros((1, _S_COLS), jnp.float32).at[0, :NUM_SAMPLED].set(
        jnp.log(NUM_SAMPLED * p_samp)
    )
    sidx = jnp.full((1, _S_COLS), -1, jnp.int32).at[0, :NUM_SAMPLED].set(sampled)
    pad_spread = (jnp.arange(_S_COLS, dtype=jnp.int32) * 3929) % VOCAB
    samp_gather_idx = pad_spread.at[:NUM_SAMPLED].set(sampled)

    info = plsc.get_sparse_core_info()
    nc, ns = info.num_cores, info.num_subcores
    true_f, samp_f = _make_sc_gather(nc, ns)(table_t, labels, samp_gather_idx)
    true_w = true_f.reshape(BATCH, DIM)
    samp_w = samp_f.reshape(_S_COLS, DIM)

    loss = pl.pallas_call(
        _tc_body,
        out_shape=jax.ShapeDtypeStruct((BATCH, 1), jnp.float32),
    )(user, true_w, samp_w, labels.reshape(BATCH, 1), sidx, slog)
    return loss


# speedup vs baseline: 8.1838x; 1.0258x over previous
import functools

import jax
import jax.numpy as jnp
from jax import lax
from jax.experimental import pallas as pl
from jax.experimental.pallas import tpu as pltpu
from jax.experimental.pallas import tpu_sc as plsc

NUM_SAMPLED = 100
VOCAB = 1000000
DIM = 32
BATCH = 4096

_S_COLS = 128
_LANES = 128
_CH = 8


def _log_uniform_prob(classes_f32, range_max):
    return (jnp.log(classes_f32 + 2.0) - jnp.log(classes_f32 + 1.0)) / jnp.log(
        range_max + 1.0
    )


def _make_sc_gather(nc, ns):
    nw = nc * ns
    lab_per_w = BATCH // nw
    n_bufs = 3 * _CH
    mesh = plsc.VectorSubcoreMesh(core_axis_name="c", subcore_axis_name="s")

    @functools.partial(
        pl.kernel,
        mesh=mesh,
        compiler_params=pltpu.CompilerParams(needs_layout_passes=False),
        out_type=(
            jax.ShapeDtypeStruct((BATCH * DIM,), jnp.float32),
            jax.ShapeDtypeStruct((_S_COLS * DIM,), jnp.float32),
        ),
        scratch_types=[
            pltpu.VMEM((lab_per_w + 16,), jnp.int32),
            pltpu.VMEM((n_bufs * DIM, _LANES), jnp.float32),
            pltpu.VMEM((lab_per_w * DIM,), jnp.float32),
            pltpu.VMEM((4 * DIM,), jnp.float32),
            pltpu.SemaphoreType.DMA,
        ],
    )
    def sc_gather(table_t, labels_hbm, sidx_hbm, true_out, samp_out,
                  idx_vmem, ring, rows_l, rows_s, sem):
        cid = lax.axis_index("c")
        sid = lax.axis_index("s")
        wid = sid * nc + cid
        base_l = pl.multiple_of(wid * lab_per_w, 8)
        pltpu.sync_copy(
            labels_hbm.at[pl.ds(base_l, lab_per_w)],
            idx_vmem.at[pl.ds(0, lab_per_w)],
        )

        iota = jax.lax.iota(jnp.int32, 16)
        mask_one = iota == 0
        int_min = jnp.int32(-2147483648)

        def pick(vec, t):
            return jnp.max(jnp.where(iota == t, vec, int_min))

        def fire(labvec, t, buf):
            col = pick(labvec, t)
            cbase = pl.multiple_of((col // _LANES) * _LANES, _LANES)
            pltpu.async_copy(
                table_t.at[:, pl.ds(cbase, _LANES)],
                ring.at[pl.ds(buf * DIM, DIM)],
                sem,
            )

        def drain_one(buf):
            pltpu.make_async_copy(
                table_t.at[:, pl.ds(0, _LANES)],
                ring.at[pl.ds(buf * DIM, DIM)],
                sem,
            ).wait()

        def extract_chunk(labvec, rows, chunk, phase):
            lanevec = labvec - (labvec // _LANES) * _LANES
            for t in range(_CH):
                lvec = jnp.take(lanevec, jnp.full((16,), t, jnp.int32))
                svec = jnp.full((16,), (chunk * _CH + t) * DIM, jnp.int32)
                buf_row = (phase + t) * DIM
                for d in range(DIM):
                    vals = plsc.load_gather(ring.at[buf_row + d], [lvec])
                    plsc.store_scatter(rows, [svec + d], vals, mask=mask_one)

        def load_labvec(c):
            off = pl.multiple_of(c * _CH, 8)
            return idx_vmem[pl.ds(off, 16)]

        def gather_cols(rows, n_slots):
            n_chunks = n_slots // _CH

            def chunk_body(c, carry):
                phase = (c % 3) * _CH
                prev_phase = ((c - 2) % 3) * _CH
                labvec = load_labvec(c)
                for t in range(_CH):
                    fire(labvec, t, phase + t)

                @pl.when(c > 1)
                def _():
                    for t in range(_CH):
                        drain_one(prev_phase + t)
                    extract_chunk(load_labvec(c - 2), rows, c - 2, prev_phase)

                return carry

            lax.fori_loop(0, n_chunks, chunk_body, 0)
            for last in (n_chunks - 2, n_chunks - 1):
                last_phase = (last % 3) * _CH
                for t in range(_CH):
                    drain_one(last_phase + t)
                extract_chunk(load_labvec(last), rows, last, last_phase)

        gather_cols(rows_l, lab_per_w)
        out_base = pl.multiple_of(wid * lab_per_w * DIM, 8)
        pltpu.sync_copy(rows_l, true_out.at[pl.ds(out_base, lab_per_w * DIM)])

        sbase = pl.multiple_of((wid // 2) * 8, 8)
        pltpu.sync_copy(sidx_hbm.at[pl.ds(sbase, 8)], idx_vmem.at[pl.ds(0, 8)])
        svecb = idx_vmem[pl.ds(0, 16)]
        lanevec_s = svecb - (svecb // _LANES) * _LANES
        off = (wid % 2) * 4
        for t in range(4):
            fire(svecb, off + t, t)
        for t in range(4):
            drain_one(t)
        for t in range(4):
            lvec = jnp.take(lanevec_s, jnp.full((16,), off + t, jnp.int32))
            svec = jnp.full((16,), t * DIM, jnp.int32)
            for d in range(DIM):
                vals = plsc.load_gather(ring.at[t * DIM + d], [lvec])
                plsc.store_scatter(rows_s, [svec + d], vals, mask=mask_one)
        pltpu.sync_copy(
            rows_s,
            samp_out.at[pl.ds(pl.multiple_of(wid * (4 * DIM), 8), 4 * DIM)],
        )

    return sc_gather


def _tc_body(user_ref, truew_ref, sampw_ref, labels_ref, sidx_ref, slog_ref,
             out_ref):
    u = user_ref[...]
    tw = truew_ref[...]
    sw = sampw_ref[...]
    lab = labels_ref[...]
    sidx = sidx_ref[...]
    slog = slog_ref[...]

    lf = lab.astype(jnp.float32)
    p_true = _log_uniform_prob(lf, float(VOCAB))
    true_logit = (
        jnp.sum(u * tw, axis=1, keepdims=True)
        - jnp.log(NUM_SAMPLED * p_true)
    )

    s_logits = (
        lax.dot_general(u, sw, (((1,), (1,)), ((), ())),
                        preferred_element_type=jnp.float32)
        - slog
    )
    col = lax.broadcasted_iota(jnp.int32, (1, _S_COLS), 1)
    dead = (sidx == lab) | (col >= NUM_SAMPLED)
    s_logits = jnp.where(dead, jnp.float32(-1e9), s_logits)

    m = jnp.maximum(jnp.max(s_logits, axis=1, keepdims=True), true_logit)
    ssum = jnp.sum(jnp.exp(s_logits - m), axis=1, keepdims=True) + jnp.exp(
        true_logit - m
    )
    out_ref[...] = m + jnp.log(ssum) - true_logit


def kernel(item_embeddings, user_embeddings, label_idx, zero_bias):
    del zero_bias
    table_t = jnp.squeeze(item_embeddings, axis=1).T
    user = user_embeddings.reshape(BATCH, DIM)
    labels = label_idx.reshape(BATCH).astype(jnp.int32)

    skey = jax.random.key(42)
    u01 = jax.random.uniform(skey, (NUM_SAMPLED,), dtype=jnp.float32)
    sampled = jnp.clip(
        (jnp.exp(u01 * jnp.log(VOCAB + 1.0)) - 1.0).astype(labels.dtype),
        0,
        VOCAB - 1,
    )
    p_samp = _log_uniform_prob(sampled.astype(jnp.float32), float(VOCAB))
    slog = jnp.zeros((1, _S_COLS), jnp.float32).at[0, :NUM_SAMPLED].set(
        jnp.log(NUM_SAMPLED * p_samp)
    )
    sidx = jnp.full((1, _S_COLS), -1, jnp.int32).at[0, :NUM_SAMPLED].set(sampled)
    pad_spread = (jnp.arange(_S_COLS, dtype=jnp.int32) * 3929) % VOCAB
    samp_gather_idx = pad_spread.at[:NUM_SAMPLED].set(sampled)

    info = plsc.get_sparse_core_info()
    nc, ns = info.num_cores, info.num_subcores
    true_f, samp_f = _make_sc_gather(nc, ns)(table_t, labels, samp_gather_idx)
    true_w = true_f.reshape(BATCH, DIM)
    samp_w = samp_f.reshape(_S_COLS, DIM)

    loss = pl.pallas_call(
        _tc_body,
        out_shape=jax.ShapeDtypeStruct((BATCH, 1), jnp.float32),
    )(user, true_w, samp_w, labels.reshape(BATCH, 1), sidx, slog)
    return loss
